# baseline (speedup 1.0000x reference)
import functools

import jax
import jax.numpy as jnp
from jax import lax
from jax.experimental import pallas as pl
from jax.experimental.pallas import tpu as pltpu
from jax.experimental.pallas import tpu_sc as plsc

N = 10000
E = 320000
H = 128
G = 64
NUM_ATOM_FEATS = 9
VOCAB = 512 * 9

NC, NS = 2, 16
NW = NC * NS
NT = 336
NP = NW * NT
NP2 = NP // 2
ACCR = NP2 + 8
EC = 128
ECT = 160
ECHH = 80
EP = NS * ECT * EC
SAC = NP
NCH = 24
NPC = 14
NBUF = 4
R = 512
GRID = NP // R

_f32 = jnp.float32
_i32 = jnp.int32
_mesh = plsc.VectorSubcoreMesh(
    core_axis_name="c", subcore_axis_name="s", num_cores=NC, num_subcores=NS)


def _zero_rows(buf, nrows):
  def body(i, carry):
    for j in range(H // 16):
      buf[i, pl.ds(j * 16, 16)] = jnp.zeros((16,), _f32)
    return carry
  lax.fori_loop(0, nrows, body, 0)


def _zero_local(zbuf, acc, base):
  for k in range(NT // 112):
    pltpu.sync_copy(zbuf.at[pl.ds(0, 112)], acc.at[pl.ds(base + k * 112, 112)])


def _remap_half(dst_v, c):
  def row(q, carry):
    for kk in range(EC // 16):
      v = dst_v[q, pl.ds(kk * 16, 16)]
      t = v - c * NP2
      ok = (t >= 0) & (t < NP2)
      dst_v[q, pl.ds(kk * 16, 16)] = jnp.where(ok, t, NP2)
    return carry
  lax.fori_loop(0, ECHH, row, 0)


def _stage_out(acc, out_at, stage, src_base, dst_base, nrows):
  def wout(k, carry):
    pltpu.sync_copy(acc.at[pl.ds(src_base + k * EC, EC)], stage)
    pltpu.sync_copy(stage, out_at(pl.ds(dst_base + k * EC, EC)))
    return carry
  lax.fori_loop(0, nrows // EC, wout, 0)
  rem = nrows - nrows // EC * EC
  if rem:
    st = stage.at[pl.ds(0, rem)]
    pltpu.sync_copy(acc.at[pl.ds(src_base + nrows // EC * EC, rem)], st)
    pltpu.sync_copy(st, out_at(pl.ds(dst_base + nrows // EC * EC, rem)))


@functools.partial(
    pl.kernel,
    out_type=(jax.ShapeDtypeStruct((NP, H), _f32),
              jax.ShapeDtypeStruct((NP, H), _f32)),
    mesh=_mesh,
    scratch_types=(
        pltpu.VMEM_SHARED((ACCR, H), _f32),
        pltpu.VMEM((NCH, 126), _i32),
        pltpu.VMEM((NCH, 126), _i32),
        pltpu.VMEM((EC, H), _f32),
        pltpu.VMEM((EC, H), _f32),
        pltpu.VMEM((EC, H), _f32),
        pltpu.VMEM((EC, H), _f32),
        pltpu.VMEM((ECHH, EC), _i32),
        (pltpu.SemaphoreType.DMA,) * NBUF,
        (pltpu.SemaphoreType.DMA,) * NBUF,
    ),
)
def _sc_emb_deg(table_h, xi_h, scat_h, dst_h, h0_o, deg_o,
                acc, xi_v, scat_v, r0, r1, r2, r3, dst_v, sg, ss):
  c = lax.axis_index("c")
  s = lax.axis_index("s")
  wid = c * NS + s
  base = s * NT
  bufs = (r0, r1, r2, r3)

  _zero_rows(r0, 112)
  _zero_local(r0, acc, base)
  pltpu.sync_copy(xi_h.at[wid], xi_v)
  pltpu.sync_copy(scat_h.at[wid], scat_v)
  plsc.subcore_barrier()

  def fill1(i, carry):
    for j in range(H // 16):
      r0[i, pl.ds(j * 16, 16)] = jnp.ones((16,), _f32)
    return carry
  lax.fori_loop(0, EC, fill1, 0)
  for half in range(2):
    pltpu.sync_copy(dst_h.at[s, pl.ds(half * ECHH, ECHH)], dst_v)
    _remap_half(dst_v, c)
    def grp(g, carry):
      for u in range(8):
        pltpu.async_copy(r0, acc.at[dst_v.at[g * 8 + u]], ss[u % NBUF],
                         add=True)
      for u in range(8):
        pltpu.make_async_copy(r0, acc.at[pl.ds(0, EC)], ss[u % NBUF]).wait()
      return carry
    lax.fori_loop(0, ECHH // 8, grp, 0)
  plsc.subcore_barrier()
  _stage_out(acc, lambda sl: deg_o.at[sl], r1, base, wid * NT, NT)
  plsc.subcore_barrier()
  _zero_rows(r0, 112)
  _zero_local(r0, acc, base)
  plsc.subcore_barrier()

  sts = tuple(b.at[pl.ds(0, 126)] for b in bufs)
  pltpu.async_copy(table_h.at[xi_v.at[0]], sts[0], sg[0])
  pltpu.async_copy(table_h.at[xi_v.at[1]], sts[1], sg[1])
  def estep(ko, carry):
    for u in range(NBUF):
      k = ko * NBUF + u
      b2 = (u + 2) % NBUF
      @pl.when(k >= 2)
      def _():
        pltpu.make_async_copy(sts[b2], acc.at[pl.ds(0, 126)], ss[b2]).wait()
      @pl.when(k + 2 < NCH)
      def _():
        pltpu.async_copy(table_h.at[xi_v.at[k + 2]], sts[b2], sg[b2])
      pltpu.make_async_copy(table_h.at[xi_v.at[k]], sts[u], sg[u]).wait()
      pltpu.async_copy(sts[u], acc.at[scat_v.at[k]], ss[u], add=True)
    return carry
  lax.fori_loop(0, NCH // NBUF, estep, 0)
  pltpu.make_async_copy(sts[2], acc.at[pl.ds(0, 126)], ss[2]).wait()
  pltpu.make_async_copy(sts[3], acc.at[pl.ds(0, 126)], ss[3]).wait()
  plsc.subcore_barrier()
  _stage_out(acc, lambda sl: h0_o.at[sl], r0, base, wid * NT, NT)


@functools.partial(
    pl.kernel,
    out_type=jax.ShapeDtypeStruct((NP, H), _f32),
    mesh=_mesh,
    scratch_types=(
        pltpu.VMEM_SHARED((ACCR, H), _f32),
        pltpu.VMEM((ECHH, EC), _i32),
        pltpu.VMEM((ECHH, EC), _i32),
        pltpu.VMEM((EC, H), _f32),
        pltpu.VMEM((EC, H), _f32),
        pltpu.VMEM((EC, H), _f32),
        pltpu.VMEM((EC, H), _f32),
        (pltpu.SemaphoreType.DMA,) * NBUF,
        (pltpu.SemaphoreType.DMA,) * NBUF,
    ),
)
def _sc_aggregate(u_h, src_h, dst_h, part_o,
                  acc, src_v, dst_v, r0, r1, r2, r3, sg, ss):
  c = lax.axis_index("c")
  s = lax.axis_index("s")
  wid = c * NS + s
  base = s * NT
  bufs = (r0, r1, r2, r3)

  _zero_rows(r0, 112)
  _zero_local(r0, acc, base)
  plsc.subcore_barrier()

  for half in range(2):
    pltpu.sync_copy(src_h.at[s, pl.ds(half * ECHH, ECHH)], src_v)
    pltpu.sync_copy(dst_h.at[s, pl.ds(half * ECHH, ECHH)], dst_v)
    _remap_half(dst_v, c)
    pltpu.async_copy(u_h.at[src_v.at[0]], bufs[0], sg[0])
    pltpu.async_copy(u_h.at[src_v.at[1]], bufs[1], sg[1])
    def step(jo, carry):
      for u in range(NBUF):
        j = jo * NBUF + u
        b2 = (u + 2) % NBUF
        @pl.when(j >= 2)
        def _():
          pltpu.make_async_copy(bufs[b2], acc.at[pl.ds(0, EC)], ss[b2]).wait()
        @pl.when(j + 2 < ECHH)
        def _():
          pltpu.async_copy(u_h.at[src_v.at[j + 2]], bufs[b2], sg[b2])
        pltpu.make_async_copy(u_h.at[src_v.at[j]], bufs[u], sg[u]).wait()
        pltpu.async_copy(bufs[u], acc.at[dst_v.at[j]], ss[u], add=True)
      return carry
    lax.fori_loop(0, ECHH // NBUF, step, 0)
    pltpu.make_async_copy(bufs[2], acc.at[pl.ds(0, EC)], ss[2]).wait()
    pltpu.make_async_copy(bufs[3], acc.at[pl.ds(0, EC)], ss[3]).wait()

  plsc.subcore_barrier()
  _stage_out(acc, lambda sl: part_o.at[sl], r0, base, wid * NT, NT)


def _dis_block(d_ref):
  return lax.rsqrt(d_ref[:, 0:1] + 1.0)


def _tc_first_body(h_ref, d_ref, w_ref, o_ref):
  dis = _dis_block(d_ref)
  o_ref[...] = jnp.dot(h_ref[...] * dis, w_ref[...],
                       preferred_element_type=_f32)


def _tc_mid_body(s_ref, u_ref, d_ref, b_ref, w_ref, o_ref):
  dis = _dis_block(d_ref)
  h = jnp.maximum(dis * (s_ref[...] + u_ref[...]) + b_ref[...], 0.0)
  o_ref[...] = jnp.dot(h * dis, w_ref[...], preferred_element_type=_f32)


def _tc_final_body(s_ref, u_ref, d_ref, b_ref, bi_ref,
                   l1w_ref, l1b_ref, w2_ref, b2_ref, o_ref, acc, cnt):
  i = pl.program_id(0)

  @pl.when(i == 0)
  def _():
    acc[...] = jnp.zeros((G, H), _f32)
    cnt[...] = jnp.zeros((G, H), _f32)

  dis = _dis_block(d_ref)
  h = jnp.maximum(dis * (s_ref[...] + u_ref[...]) + b_ref[...], 0.0)
  iota_g = lax.broadcasted_iota(_i32, (G, H), 0)
  a = acc[...]
  ct = cnt[...]
  for r in range(R // H):
    oh = (iota_g == bi_ref[0, r, :][None, :]).astype(_f32)
    a = a + jnp.dot(oh, h[r * H:(r + 1) * H, :], preferred_element_type=_f32)
    ct = ct + jnp.sum(oh, axis=1, keepdims=True)
  acc[...] = a
  cnt[...] = ct

  @pl.when(i == GRID - 1)
  def _():
    pooled = acc[...] / jnp.maximum(cnt[...], 1.0)
    z = jnp.maximum(
        jnp.dot(pooled, l1w_ref[...], preferred_element_type=_f32)
        + l1b_ref[...], 0.0)
    rsum = jnp.sum(z * w2_ref[...], axis=1, keepdims=True)
    o_ref[...] = jnp.broadcast_to(rsum, (G, H)) + b2_ref[...]


_row_spec = pl.BlockSpec((R, H), lambda i: (i, 0))
_w_spec = pl.BlockSpec((H, H), lambda i: (0, 0))
_vec_spec = pl.BlockSpec((1, H), lambda i: (0, 0))

_tc_first = pl.pallas_call(
    _tc_first_body,
    grid=(GRID,),
    in_specs=[_row_spec, _row_spec, _w_spec],
    out_specs=_row_spec,
    out_shape=jax.ShapeDtypeStruct((NP, H), _f32),
)

_tc_mid = pl.pallas_call(
    _tc_mid_body,
    grid=(GRID,),
    in_specs=[_row_spec, _row_spec, _row_spec, _vec_spec, _w_spec],
    out_specs=_row_spec,
    out_shape=jax.ShapeDtypeStruct((NP, H), _f32),
)

_tc_final = pl.pallas_call(
    _tc_final_body,
    grid=(GRID,),
    in_specs=[_row_spec, _row_spec, _row_spec, _vec_spec,
              pl.BlockSpec((1, R // H, H), lambda i: (i, 0, 0)),
              _w_spec, _vec_spec, _vec_spec, _vec_spec],
    out_specs=pl.BlockSpec((G, H), lambda i: (0, 0)),
    out_shape=jax.ShapeDtypeStruct((G, H), _f32),
    scratch_shapes=[pltpu.VMEM((G, H), _f32), pltpu.VMEM((G, H), _f32)],
)


def kernel(x, edge_index, edge_attr, batch_idx, table,
           W1, b1, W2, b2, W3, b3, lin1_W, lin1_b, lin2_W, lin2_b):
  del edge_attr

  offsets = (1 + jnp.arange(0, NUM_ATOM_FEATS * 512, 512)).astype(_i32)
  xi = jnp.clip(x.astype(_i32) + offsets[None, :], 0, VOCAB - 1)
  xi_blk = jnp.pad(xi, ((0, NP - N), (0, 0))).reshape(NW, NCH, NPC * 9)
  scat_blk = (jnp.broadcast_to(
      jnp.arange(NP, dtype=_i32)[:, None], (NP, NUM_ATOM_FEATS)
  ) % NP2).reshape(NW, NCH, NPC * 9)

  src = edge_index[0].astype(_i32)
  dst = edge_index[1].astype(_i32)
  src_blk = jnp.pad(src, (0, EP - E)).reshape(NS, ECT, EC)
  dst_blk = jnp.pad(dst, (0, EP - E), constant_values=SAC).reshape(NS, ECT, EC)

  bi_blk = jnp.pad(batch_idx.astype(_i32), (0, NP - N),
                   constant_values=G).reshape(GRID, R // H, H)

  h0, deg = _sc_emb_deg(table, xi_blk, scat_blk, dst_blk)

  b1r = b1.reshape(1, H)
  b2r = b2.reshape(1, H)
  b3r = b3.reshape(1, H)
  l1br = lin1_b.reshape(1, H)
  w2r = lin2_W.reshape(1, H)
  b2sr = jnp.broadcast_to(lin2_b.reshape(1, 1), (1, H))

  u1 = _tc_first(h0, deg, W1)
  s1 = _sc_aggregate(u1, src_blk, dst_blk)
  u2 = _tc_mid(s1, u1, deg, b1r, W2)
  s2 = _sc_aggregate(u2, src_blk, dst_blk)
  u3 = _tc_mid(s2, u2, deg, b2r, W3)
  s3 = _sc_aggregate(u3, src_blk, dst_blk)
  res = _tc_final(s3, u3, deg, b3r, bi_blk, lin1_W, l1br, w2r, b2sr)
  return res[:, :1]


# speedup vs baseline: 1.6831x; 1.6831x over previous
import functools

import jax
import jax.numpy as jnp
from jax import lax
from jax.experimental import pallas as pl
from jax.experimental.pallas import tpu as pltpu
from jax.experimental.pallas import tpu_sc as plsc

N = 10000
E = 320000
H = 128
G = 64
NUM_ATOM_FEATS = 9
VOCAB = 512 * 9

NC, NS = 2, 16
NW = NC * NS
NT = 336
NP = NW * NT
RPT = NP // NS
EC = 128
ECH = 80
ECHH = ECH // 2
EP = NW * ECH * EC
SAC = N
NCH = 24
NPC = 14
ZR = 32
R = 512
GRID = NP // R

_f32 = jnp.float32
_i32 = jnp.int32
_mesh = plsc.VectorSubcoreMesh(
    core_axis_name="c", subcore_axis_name="s", num_cores=NC, num_subcores=NS)


def _zero_rows(buf, nrows):
  def body(i, carry):
    for j in range(H // 16):
      buf[i, pl.ds(j * 16, 16)] = jnp.zeros((16,), _f32)
    return carry
  lax.fori_loop(0, nrows, body, 0)


def _zero_acc(zb, acc, base):
  def body(k, carry):
    pltpu.sync_copy(zb, acc.at[pl.ds(base + k * ZR, ZR)])
    return carry
  lax.fori_loop(0, RPT // ZR, body, 0)


def _stage_out(acc, out_at, stage, src_base, dst_base, nrows):
  def wout(k, carry):
    pltpu.sync_copy(acc.at[pl.ds(src_base + k * EC, EC)], stage)
    pltpu.sync_copy(stage, out_at(pl.ds(dst_base + k * EC, EC)))
    return carry
  lax.fori_loop(0, nrows // EC, wout, 0)
  rem = nrows - nrows // EC * EC
  if rem:
    st = stage.at[pl.ds(0, rem)]
    pltpu.sync_copy(acc.at[pl.ds(src_base + nrows // EC * EC, rem)], st)
    pltpu.sync_copy(st, out_at(pl.ds(dst_base + nrows // EC * EC, rem)))


@functools.partial(
    pl.kernel,
    out_type=(jax.ShapeDtypeStruct((NP, H), _f32),
              jax.ShapeDtypeStruct((NC, NP, H), _f32)),
    mesh=_mesh,
    scratch_types=(
        pltpu.VMEM_SHARED((NP, H), _f32),
        pltpu.VMEM((ZR, H), _f32),
        pltpu.VMEM((NCH, 126), _i32),
        pltpu.VMEM((NCH, 126), _i32),
        pltpu.VMEM((EC, H), _f32),
        pltpu.VMEM((ECHH, EC), _i32),
        pltpu.SemaphoreType.DMA,
        pltpu.SemaphoreType.DMA,
    ),
)
def _sc_emb_deg(table_h, xi_h, scat_h, dst_h, h0_o, deg_o,
                acc, zb, xi_v, scat_v, rows_v, dst_v, sem_a, sem_b):
  c = lax.axis_index("c")
  s = lax.axis_index("s")
  wid = s * NC + c

  _zero_rows(zb, ZR)
  def fill1(i, carry):
    for j in range(H // 16):
      rows_v[i, pl.ds(j * 16, 16)] = jnp.ones((16,), _f32)
    return carry
  lax.fori_loop(0, EC, fill1, 0)

  _zero_acc(zb, acc, s * RPT)
  pltpu.sync_copy(xi_h.at[wid], xi_v)
  pltpu.sync_copy(scat_h.at[wid], scat_v)
  plsc.subcore_barrier()

  for half in range(2):
    pltpu.sync_copy(dst_h.at[wid, pl.ds(half * ECHH, ECHH)], dst_v)
    def grp(g, carry):
      for u in range(4):
        pltpu.async_copy(rows_v, acc.at[dst_v.at[g * 8 + 2 * u]], sem_a,
                         add=True)
        pltpu.async_copy(rows_v, acc.at[dst_v.at[g * 8 + 2 * u + 1]], sem_b,
                         add=True)
      for u in range(4):
        pltpu.make_async_copy(rows_v, acc.at[pl.ds(0, EC)], sem_a).wait()
        pltpu.make_async_copy(rows_v, acc.at[pl.ds(0, EC)], sem_b).wait()
      return carry
    lax.fori_loop(0, ECHH // 8, grp, 0)
  plsc.subcore_barrier()
  _stage_out(acc, lambda sl: deg_o.at[c, sl], rows_v, s * RPT, s * RPT, RPT)
  plsc.subcore_barrier()
  _zero_acc(zb, acc, s * RPT)
  plsc.subcore_barrier()

  def emb_chunk(k, carry):
    st = rows_v.at[pl.ds(0, 126)]
    pltpu.async_copy(table_h.at[xi_v.at[k]], st, sem_a).wait()
    pltpu.sync_copy(st, acc.at[scat_v.at[k]], add=True)
    return carry
  lax.fori_loop(0, NCH, emb_chunk, 0)
  plsc.subcore_barrier()
  _stage_out(acc, lambda sl: h0_o.at[sl], rows_v, wid * NT, wid * NT, NT)


@functools.partial(
    pl.kernel,
    out_type=jax.ShapeDtypeStruct((NC, NP, H), _f32),
    mesh=_mesh,
    scratch_types=(
        pltpu.VMEM_SHARED((NP, H), _f32),
        pltpu.VMEM((ECHH, EC), _i32),
        pltpu.VMEM((ECHH, EC), _i32),
        pltpu.VMEM((EC, H), _f32),
        pltpu.VMEM((EC, H), _f32),
        pltpu.SemaphoreType.DMA,
        pltpu.SemaphoreType.DMA,
    ),
)
def _sc_aggregate(u_h, src_h, dst_h, part_o,
                  acc, src_v, dst_v, rows_a, rows_b, sem_a, sem_b):
  c = lax.axis_index("c")
  s = lax.axis_index("s")
  wid = s * NC + c

  _zero_rows(rows_a, ZR)
  _zero_acc(rows_a.at[pl.ds(0, ZR)], acc, s * RPT)
  plsc.subcore_barrier()

  bufs = (rows_a, rows_b)
  sems = (sem_a, sem_b)
  for half in range(2):
    pltpu.sync_copy(src_h.at[wid, pl.ds(half * ECHH, ECHH)], src_v)
    pltpu.sync_copy(dst_h.at[wid, pl.ds(half * ECHH, ECHH)], dst_v)
    pltpu.async_copy(u_h.at[src_v.at[0]], bufs[0], sems[0])
    def agg_pair(jj, carry):
      j0 = jj * 2
      pltpu.async_copy(u_h.at[src_v.at[j0 + 1]], bufs[1], sems[1])
      pltpu.make_async_copy(u_h.at[src_v.at[j0]], bufs[0], sems[0]).wait()
      pltpu.sync_copy(bufs[0], acc.at[dst_v.at[j0]], add=True)
      @pl.when(jj < ECHH // 2 - 1)
      def _():
        pltpu.async_copy(u_h.at[src_v.at[j0 + 2]], bufs[0], sems[0])
      pltpu.make_async_copy(u_h.at[src_v.at[j0 + 1]], bufs[1], sems[1]).wait()
      pltpu.sync_copy(bufs[1], acc.at[dst_v.at[j0 + 1]], add=True)
      return carry
    lax.fori_loop(0, ECHH // 2, agg_pair, 0)

  plsc.subcore_barrier()
  _stage_out(acc, lambda sl: part_o.at[c, sl], rows_a, s * RPT, s * RPT, RPT)


def _dis_block(d0_ref, d1_ref):
  deg = d0_ref[:, 0:1] + d1_ref[:, 0:1] + 1.0
  return lax.rsqrt(deg)


def _tc_first_body(h_ref, d0_ref, d1_ref, w_ref, o_ref):
  dis = _dis_block(d0_ref, d1_ref)
  o_ref[...] = jnp.dot(h_ref[...] * dis, w_ref[...],
                       preferred_element_type=_f32)


def _tc_mid_body(p0_ref, p1_ref, u_ref, d0_ref, d1_ref, b_ref, w_ref, o_ref):
  dis = _dis_block(d0_ref, d1_ref)
  h = jnp.maximum(dis * (p0_ref[...] + p1_ref[...] + u_ref[...]) + b_ref[...],
                  0.0)
  o_ref[...] = jnp.dot(h * dis, w_ref[...], preferred_element_type=_f32)


def _tc_final_body(p0_ref, p1_ref, u_ref, d0_ref, d1_ref, b_ref, bi_ref,
                   l1w_ref, l1b_ref, w2_ref, b2_ref, o_ref, acc, cnt):
  i = pl.program_id(0)

  @pl.when(i == 0)
  def _():
    acc[...] = jnp.zeros((G, H), _f32)
    cnt[...] = jnp.zeros((G, H), _f32)

  dis = _dis_block(d0_ref, d1_ref)
  h = jnp.maximum(dis * (p0_ref[...] + p1_ref[...] + u_ref[...]) + b_ref[...],
                  0.0)
  iota_g = lax.broadcasted_iota(_i32, (G, H), 0)
  a = acc[...]
  ct = cnt[...]
  for r in range(R // H):
    oh = (iota_g == bi_ref[0, r, :][None, :]).astype(_f32)
    a = a + jnp.dot(oh, h[r * H:(r + 1) * H, :], preferred_element_type=_f32)
    ct = ct + jnp.sum(oh, axis=1, keepdims=True)
  acc[...] = a
  cnt[...] = ct

  @pl.when(i == GRID - 1)
  def _():
    pooled = acc[...] / jnp.maximum(cnt[...], 1.0)
    z = jnp.maximum(
        jnp.dot(pooled, l1w_ref[...], preferred_element_type=_f32)
        + l1b_ref[...], 0.0)
    rsum = jnp.sum(z * w2_ref[...], axis=1, keepdims=True)
    o_ref[...] = jnp.broadcast_to(rsum, (G, H)) + b2_ref[...]


_row_spec = pl.BlockSpec((R, H), lambda i: (i, 0))
_w_spec = pl.BlockSpec((H, H), lambda i: (0, 0))
_vec_spec = pl.BlockSpec((1, H), lambda i: (0, 0))

_tc_first = pl.pallas_call(
    _tc_first_body,
    grid=(GRID,),
    in_specs=[_row_spec, _row_spec, _row_spec, _w_spec],
    out_specs=_row_spec,
    out_shape=jax.ShapeDtypeStruct((NP, H), _f32),
)

_tc_mid = pl.pallas_call(
    _tc_mid_body,
    grid=(GRID,),
    in_specs=[_row_spec, _row_spec, _row_spec, _row_spec, _row_spec,
              _vec_spec, _w_spec],
    out_specs=_row_spec,
    out_shape=jax.ShapeDtypeStruct((NP, H), _f32),
)

_tc_final = pl.pallas_call(
    _tc_final_body,
    grid=(GRID,),
    in_specs=[_row_spec, _row_spec, _row_spec, _row_spec, _row_spec,
              _vec_spec, pl.BlockSpec((1, R // H, H), lambda i: (i, 0, 0)),
              _w_spec, _vec_spec, _vec_spec, _vec_spec],
    out_specs=pl.BlockSpec((G, H), lambda i: (0, 0)),
    out_shape=jax.ShapeDtypeStruct((G, H), _f32),
    scratch_shapes=[pltpu.VMEM((G, H), _f32), pltpu.VMEM((G, H), _f32)],
)


def kernel(x, edge_index, edge_attr, batch_idx, table,
           W1, b1, W2, b2, W3, b3, lin1_W, lin1_b, lin2_W, lin2_b):
  del edge_attr

  offsets = (1 + jnp.arange(0, NUM_ATOM_FEATS * 512, 512)).astype(_i32)
  xi = jnp.clip(x.astype(_i32) + offsets[None, :], 0, VOCAB - 1)
  xi_blk = jnp.pad(xi, ((0, NP - N), (0, 0))).reshape(NW, NCH, NPC * 9)
  scat_blk = jnp.broadcast_to(
      jnp.arange(NP, dtype=_i32)[:, None], (NP, NUM_ATOM_FEATS)
  ).reshape(NW, NCH, NPC * 9)

  src = edge_index[0].astype(_i32)
  dst = edge_index[1].astype(_i32)
  src_blk = jnp.pad(src, (0, EP - E)).reshape(NW, ECH, EC)
  dst_blk = jnp.pad(dst, (0, EP - E), constant_values=SAC).reshape(NW, ECH, EC)

  bi_blk = jnp.pad(batch_idx.astype(_i32), (0, NP - N),
                   constant_values=G).reshape(GRID, R // H, H)

  h0, deg = _sc_emb_deg(table, xi_blk, scat_blk, dst_blk)
  d0, d1 = deg[0], deg[1]

  b1r = b1.reshape(1, H)
  b2r = b2.reshape(1, H)
  b3r = b3.reshape(1, H)
  l1br = lin1_b.reshape(1, H)
  w2r = lin2_W.reshape(1, H)
  b2sr = jnp.broadcast_to(lin2_b.reshape(1, 1), (1, H))

  u1 = _tc_first(h0, d0, d1, W1)
  s1 = _sc_aggregate(u1, src_blk, dst_blk)
  u2 = _tc_mid(s1[0], s1[1], u1, d0, d1, b1r, W2)
  s2 = _sc_aggregate(u2, src_blk, dst_blk)
  u3 = _tc_mid(s2[0], s2[1], u2, d0, d1, b2r, W3)
  s3 = _sc_aggregate(u3, src_blk, dst_blk)
  res = _tc_final(s3[0], s3[1], u3, d0, d1, b3r, bi_blk,
                  lin1_W, l1br, w2r, b2sr)
  return res[:, :1]
